# baseline (speedup 1.0000x reference)
import functools

import jax
import jax.numpy as jnp
from jax import lax
from jax.experimental import pallas as pl
from jax.experimental.pallas import tpu as pltpu
from jax.experimental.pallas import tpu_sc as plsc

_D = 32
_L = 16
_SR = 128


@jax.jit
def _run(champ1, champ2, table):
    B = champ1.shape[0]
    n_champ = table.shape[0]
    info = plsc.get_sparse_core_info()
    nw = info.num_cores * info.num_subcores
    b_per_w = B // nw
    half = b_per_w // 2

    t128 = table.reshape(n_champ // 4, _SR)

    mesh = plsc.VectorSubcoreMesh(core_axis_name="c", subcore_axis_name="s")

    @functools.partial(
        pl.kernel,
        mesh=mesh,
        compiler_params=pltpu.CompilerParams(needs_layout_passes=False),
        out_type=jax.ShapeDtypeStruct((B,), jnp.float32),
        scratch_types=[
            pltpu.VMEM((b_per_w,), jnp.int32),
            pltpu.VMEM((b_per_w,), jnp.int32),
            pltpu.VMEM((half,), jnp.int32),
            pltpu.VMEM((half,), jnp.int32),
            pltpu.VMEM((half, _SR), jnp.float32),
            pltpu.VMEM((half, _SR), jnp.float32),
            pltpu.VMEM((b_per_w,), jnp.float32),
            pltpu.SemaphoreType.DMA,
        ],
    )
    def k(c1_hbm, c2_hbm, t_hbm, out_hbm,
          idx1_v, idx2_v, sr1_v, sr2_v, r1_v, r2_v, out_v, sem):
        wid = lax.axis_index("s") * info.num_cores + lax.axis_index("c")
        base = wid * b_per_w
        pltpu.sync_copy(c1_hbm.at[pl.ds(base, b_per_w)], idx1_v)
        pltpu.sync_copy(c2_hbm.at[pl.ds(base, b_per_w)], idx2_v)

        lane = lax.iota(jnp.int32, _L)
        n_groups = half // _L

        for h in range(2):
            hbase = h * half

            def srow(g, carry):
                src = pl.ds(hbase + g * _L, _L)
                dst = pl.ds(g * _L, _L)
                sr1_v[dst] = jax.lax.shift_right_logical(idx1_v[src], 2)
                sr2_v[dst] = jax.lax.shift_right_logical(idx2_v[src], 2)
                return carry

            lax.fori_loop(0, n_groups, srow, 0)

            cp1 = pltpu.async_copy(t_hbm.at[sr1_v], r1_v, sem)
            cp2 = pltpu.async_copy(t_hbm.at[sr2_v], r2_v, sem)
            cp1.wait()
            cp2.wait()

            def group(g, carry):
                src = pl.ds(hbase + g * _L, _L)
                rows = lane + g * _L
                o1 = jax.lax.shift_left(idx1_v[src] & 3, 5)
                o2 = jax.lax.shift_left(idx2_v[src] & 3, 5)
                acc = jnp.zeros((_L,), jnp.float32)
                for d in range(_D):
                    a = plsc.load_gather(r1_v, [rows, o1 + d])
                    b = plsc.load_gather(r2_v, [rows, o2 + d])
                    acc = acc + a * b
                out_v[src] = acc
                return carry

            lax.fori_loop(0, n_groups, group, 0)

        pltpu.sync_copy(out_v, out_hbm.at[pl.ds(base, b_per_w)])

    return k(champ1.astype(jnp.int32), champ2.astype(jnp.int32), t128)


def kernel(champ1, champ2, table):
    return _run(champ1, champ2, table).reshape(-1, 1, 1)


# speedup vs baseline: 1.7387x; 1.7387x over previous
import functools

import jax
import jax.numpy as jnp
from jax import lax
from jax.experimental import pallas as pl
from jax.experimental.pallas import tpu as pltpu
from jax.experimental.pallas import tpu_sc as plsc

_D = 32
_L = 16
_SR = 128
_QS = 262144
_CB = 2048


def _relayout_block(x0_ref, x1_ref, x2_ref, x3_ref, out_ref):
    out_ref[:, 0:32] = x0_ref[...].T
    out_ref[:, 32:64] = x1_ref[...].T
    out_ref[:, 64:96] = x2_ref[...].T
    out_ref[:, 96:128] = x3_ref[...].T


def _tc_relayout(t2):
    nb = _QS // _CB
    last = (t2.shape[1] + _CB - 1) // _CB - 1

    def in_spec(q):
        return pl.BlockSpec(
            (_D, _CB), lambda i, q=q: (0, jnp.minimum(q * nb + i, last))
        )

    return pl.pallas_call(
        _relayout_block,
        grid=(nb,),
        in_specs=[in_spec(0), in_spec(1), in_spec(2), in_spec(3)],
        out_specs=pl.BlockSpec((_CB, _SR), lambda i: (i, 0)),
        out_shape=jax.ShapeDtypeStruct((_QS, _SR), jnp.float32),
    )(t2, t2, t2, t2)


@jax.jit
def _run(champ1, champ2, table):
    B = champ1.shape[0]
    info = plsc.get_sparse_core_info()
    nw = info.num_cores * info.num_subcores
    b_per_w = B // nw
    half = b_per_w // 2

    t128 = _tc_relayout(table.T)

    mesh = plsc.VectorSubcoreMesh(core_axis_name="c", subcore_axis_name="s")

    @functools.partial(
        pl.kernel,
        mesh=mesh,
        compiler_params=pltpu.CompilerParams(needs_layout_passes=False),
        out_type=jax.ShapeDtypeStruct((B,), jnp.float32),
        scratch_types=[
            pltpu.VMEM((b_per_w,), jnp.int32),
            pltpu.VMEM((b_per_w,), jnp.int32),
            pltpu.VMEM((half,), jnp.int32),
            pltpu.VMEM((half,), jnp.int32),
            pltpu.VMEM((half, _SR), jnp.float32),
            pltpu.VMEM((half, _SR), jnp.float32),
            pltpu.VMEM((b_per_w,), jnp.float32),
            pltpu.SemaphoreType.DMA,
        ],
    )
    def k(c1_hbm, c2_hbm, t_hbm, out_hbm,
          idx1_v, idx2_v, sr1_v, sr2_v, r1_v, r2_v, out_v, sem):
        wid = lax.axis_index("s") * info.num_cores + lax.axis_index("c")
        base = wid * b_per_w
        pltpu.sync_copy(c1_hbm.at[pl.ds(base, b_per_w)], idx1_v)
        pltpu.sync_copy(c2_hbm.at[pl.ds(base, b_per_w)], idx2_v)

        lane = lax.iota(jnp.int32, _L)
        n_groups = half // _L

        for h in range(2):
            hbase = h * half

            def srow(g, carry):
                src = pl.ds(hbase + g * _L, _L)
                dst = pl.ds(g * _L, _L)
                sr1_v[dst] = idx1_v[src] & (_QS - 1)
                sr2_v[dst] = idx2_v[src] & (_QS - 1)
                return carry

            lax.fori_loop(0, n_groups, srow, 0)

            cp1 = pltpu.async_copy(t_hbm.at[sr1_v], r1_v, sem)
            cp2 = pltpu.async_copy(t_hbm.at[sr2_v], r2_v, sem)
            cp1.wait()
            cp2.wait()

            def group(g, carry):
                src = pl.ds(hbase + g * _L, _L)
                rows = lane + g * _L
                o1 = jax.lax.shift_left(
                    jax.lax.shift_right_logical(idx1_v[src], 18), 5)
                o2 = jax.lax.shift_left(
                    jax.lax.shift_right_logical(idx2_v[src], 18), 5)
                acc = jnp.zeros((_L,), jnp.float32)
                for d in range(_D):
                    a = plsc.load_gather(r1_v, [rows, o1 + d])
                    b = plsc.load_gather(r2_v, [rows, o2 + d])
                    acc = acc + a * b
                out_v[src] = acc
                return carry

            lax.fori_loop(0, n_groups, group, 0)

        pltpu.sync_copy(out_v, out_hbm.at[pl.ds(base, b_per_w)])

    return k(champ1.astype(jnp.int32), champ2.astype(jnp.int32), t128)


def kernel(champ1, champ2, table):
    return _run(champ1, champ2, table).reshape(-1, 1, 1)


# speedup vs baseline: 4.1547x; 2.3896x over previous
import functools

import jax
import jax.numpy as jnp
from jax import lax
from jax.experimental import pallas as pl
from jax.experimental.pallas import tpu as pltpu
from jax.experimental.pallas import tpu_sc as plsc

_D = 32
_L = 16
_SR = 128
_QS = 262144
_CB = 8192


def _relayout_block(x0_ref, x1_ref, x2_ref, x3_ref, out_ref):
    x = jnp.concatenate(
        [x0_ref[...], x1_ref[...], x2_ref[...], x3_ref[...]], axis=0)
    out_ref[...] = x.T


def _tc_relayout(t2):
    nb = _QS // _CB
    last = (t2.shape[1] + _CB - 1) // _CB - 1

    def in_spec(q):
        return pl.BlockSpec(
            (_D, _CB), lambda i, q=q: (0, jnp.minimum(q * nb + i, last))
        )

    return pl.pallas_call(
        _relayout_block,
        grid=(nb,),
        in_specs=[in_spec(0), in_spec(1), in_spec(2), in_spec(3)],
        out_specs=pl.BlockSpec((_CB, _SR), lambda i: (i, 0)),
        out_shape=jax.ShapeDtypeStruct((_QS, _SR), jnp.float32),
    )(t2, t2, t2, t2)


@jax.jit
def _run(champ1, champ2, table):
    B = champ1.shape[0]
    info = plsc.get_sparse_core_info()
    nw = info.num_cores * info.num_subcores
    b_per_w = B // nw
    half = b_per_w // 2

    t128 = _tc_relayout(table.T)

    mesh = plsc.VectorSubcoreMesh(core_axis_name="c", subcore_axis_name="s")

    @functools.partial(
        pl.kernel,
        mesh=mesh,
        compiler_params=pltpu.CompilerParams(needs_layout_passes=False),
        out_type=jax.ShapeDtypeStruct((B,), jnp.float32),
        scratch_types=[
            pltpu.VMEM((b_per_w,), jnp.int32),
            pltpu.VMEM((b_per_w,), jnp.int32),
            pltpu.VMEM((half,), jnp.int32),
            pltpu.VMEM((half,), jnp.int32),
            pltpu.VMEM((half, _SR), jnp.float32),
            pltpu.VMEM((half, _SR), jnp.float32),
            pltpu.VMEM((b_per_w,), jnp.float32),
            pltpu.SemaphoreType.DMA,
        ],
    )
    def k(c1_hbm, c2_hbm, t_hbm, out_hbm,
          idx1_v, idx2_v, sr1_v, sr2_v, r1_v, r2_v, out_v, sem):
        wid = lax.axis_index("s") * info.num_cores + lax.axis_index("c")
        base = wid * b_per_w
        pltpu.sync_copy(c1_hbm.at[pl.ds(base, b_per_w)], idx1_v)
        pltpu.sync_copy(c2_hbm.at[pl.ds(base, b_per_w)], idx2_v)

        lane = lax.iota(jnp.int32, _L)
        n_groups = half // _L

        for h in range(2):
            hbase = h * half

            def srow(g, carry):
                src = pl.ds(hbase + g * _L, _L)
                dst = pl.ds(g * _L, _L)
                sr1_v[dst] = idx1_v[src] & (_QS - 1)
                sr2_v[dst] = idx2_v[src] & (_QS - 1)
                return carry

            lax.fori_loop(0, n_groups, srow, 0)

            cp1 = pltpu.async_copy(t_hbm.at[sr1_v], r1_v, sem)
            cp2 = pltpu.async_copy(t_hbm.at[sr2_v], r2_v, sem)
            cp1.wait()
            cp2.wait()

            def group(g, carry):
                src = pl.ds(hbase + g * _L, _L)
                rows = lane + g * _L
                o1 = jax.lax.shift_left(
                    jax.lax.shift_right_logical(idx1_v[src], 18), 5)
                o2 = jax.lax.shift_left(
                    jax.lax.shift_right_logical(idx2_v[src], 18), 5)
                acc = jnp.zeros((_L,), jnp.float32)
                for d in range(_D):
                    a = plsc.load_gather(r1_v, [rows, o1 + d])
                    b = plsc.load_gather(r2_v, [rows, o2 + d])
                    acc = acc + a * b
                out_v[src] = acc
                return carry

            lax.fori_loop(0, n_groups, group, 0)

        pltpu.sync_copy(out_v, out_hbm.at[pl.ds(base, b_per_w)])

    return k(champ1.astype(jnp.int32), champ2.astype(jnp.int32), t128)


def kernel(champ1, champ2, table):
    return _run(champ1, champ2, table).reshape(-1, 1, 1)


# speedup vs baseline: 4.2392x; 1.0203x over previous
import functools

import jax
import jax.numpy as jnp
from jax import lax
from jax.experimental import pallas as pl
from jax.experimental.pallas import tpu as pltpu
from jax.experimental.pallas import tpu_sc as plsc

_D = 32
_L = 16
_SR = 128
_QS = 262144
_CB = 16384


def _relayout_block(x0_ref, x1_ref, x2_ref, x3_ref, out_ref):
    x = jnp.concatenate(
        [x0_ref[...], x1_ref[...], x2_ref[...], x3_ref[...]], axis=0)
    out_ref[...] = x.T


def _tc_relayout(t2):
    nb = _QS // _CB
    last = (t2.shape[1] + _CB - 1) // _CB - 1

    def in_spec(q):
        return pl.BlockSpec(
            (_D, _CB), lambda i, q=q: (0, jnp.minimum(q * nb + i, last))
        )

    return pl.pallas_call(
        _relayout_block,
        grid=(nb,),
        in_specs=[in_spec(0), in_spec(1), in_spec(2), in_spec(3)],
        out_specs=pl.BlockSpec((_CB, _SR), lambda i: (i, 0)),
        out_shape=jax.ShapeDtypeStruct((_QS, _SR), jnp.float32),
    )(t2, t2, t2, t2)


@jax.jit
def _run(champ1, champ2, table):
    B = champ1.shape[0]
    info = plsc.get_sparse_core_info()
    nw = info.num_cores * info.num_subcores
    b_per_w = B // nw
    half = b_per_w // 2

    t128 = _tc_relayout(table.T)

    mesh = plsc.VectorSubcoreMesh(core_axis_name="c", subcore_axis_name="s")

    @functools.partial(
        pl.kernel,
        mesh=mesh,
        compiler_params=pltpu.CompilerParams(needs_layout_passes=False),
        out_type=jax.ShapeDtypeStruct((B,), jnp.float32),
        scratch_types=[
            pltpu.VMEM((b_per_w,), jnp.int32),
            pltpu.VMEM((b_per_w,), jnp.int32),
            pltpu.VMEM((half,), jnp.int32),
            pltpu.VMEM((half,), jnp.int32),
            pltpu.VMEM((half, _SR), jnp.float32),
            pltpu.VMEM((half, _SR), jnp.float32),
            pltpu.VMEM((b_per_w,), jnp.float32),
            pltpu.SemaphoreType.DMA,
        ],
    )
    def k(c1_hbm, c2_hbm, t_hbm, out_hbm,
          idx1_v, idx2_v, sr1_v, sr2_v, r1_v, r2_v, out_v, sem):
        wid = lax.axis_index("s") * info.num_cores + lax.axis_index("c")
        base = wid * b_per_w
        pltpu.sync_copy(c1_hbm.at[pl.ds(base, b_per_w)], idx1_v)
        pltpu.sync_copy(c2_hbm.at[pl.ds(base, b_per_w)], idx2_v)

        lane = lax.iota(jnp.int32, _L)
        n_groups = half // _L

        for h in range(2):
            hbase = h * half

            def srow(g, carry):
                src = pl.ds(hbase + g * _L, _L)
                dst = pl.ds(g * _L, _L)
                sr1_v[dst] = idx1_v[src] & (_QS - 1)
                sr2_v[dst] = idx2_v[src] & (_QS - 1)
                return carry

            lax.fori_loop(0, n_groups, srow, 0)

            cp1 = pltpu.async_copy(t_hbm.at[sr1_v], r1_v, sem)
            cp2 = pltpu.async_copy(t_hbm.at[sr2_v], r2_v, sem)
            cp1.wait()
            cp2.wait()

            def group(g, carry):
                src = pl.ds(hbase + g * _L, _L)
                rows = lane + g * _L
                o1 = jax.lax.shift_left(
                    jax.lax.shift_right_logical(idx1_v[src], 18), 5)
                o2 = jax.lax.shift_left(
                    jax.lax.shift_right_logical(idx2_v[src], 18), 5)
                acc = jnp.zeros((_L,), jnp.float32)
                for d in range(_D):
                    a = plsc.load_gather(r1_v, [rows, o1 + d])
                    b = plsc.load_gather(r2_v, [rows, o2 + d])
                    acc = acc + a * b
                out_v[src] = acc
                return carry

            lax.fori_loop(0, n_groups, group, 0)

        pltpu.sync_copy(out_v, out_hbm.at[pl.ds(base, b_per_w)])

    return k(champ1.astype(jnp.int32), champ2.astype(jnp.int32), t128)


def kernel(champ1, champ2, table):
    return _run(champ1, champ2, table).reshape(-1, 1, 1)


# speedup vs baseline: 4.4001x; 1.0379x over previous
import functools

import jax
import jax.numpy as jnp
from jax import lax
from jax.experimental import pallas as pl
from jax.experimental.pallas import tpu as pltpu
from jax.experimental.pallas import tpu_sc as plsc

_D = 32
_L = 16
_SR = 128
_QS = 262144
_CB = 16384


def _relayout_block(x0_ref, x1_ref, x2_ref, x3_ref, out_ref):
    x = jnp.concatenate(
        [x0_ref[...], x1_ref[...], x2_ref[...], x3_ref[...]], axis=0)
    out_ref[...] = x.T


def _tc_relayout(t2):
    nb = _QS // _CB
    last = (t2.shape[1] + _CB - 1) // _CB - 1

    def in_spec(q):
        return pl.BlockSpec(
            (_D, _CB), lambda i, q=q: (0, jnp.minimum(q * nb + i, last))
        )

    return pl.pallas_call(
        _relayout_block,
        grid=(nb,),
        in_specs=[in_spec(0), in_spec(1), in_spec(2), in_spec(3)],
        out_specs=pl.BlockSpec((_CB, _SR), lambda i: (i, 0)),
        out_shape=jax.ShapeDtypeStruct((_QS, _SR), jnp.float32),
    )(t2, t2, t2, t2)


@jax.jit
def _run(champ1, champ2, table):
    B = champ1.shape[0]
    info = plsc.get_sparse_core_info()
    nw = info.num_cores * info.num_subcores
    b_per_w = B // nw
    chunk = b_per_w // 4

    t128 = _tc_relayout(table.T)

    mesh = plsc.VectorSubcoreMesh(core_axis_name="c", subcore_axis_name="s")

    @functools.partial(
        pl.kernel,
        mesh=mesh,
        compiler_params=pltpu.CompilerParams(needs_layout_passes=False),
        out_type=jax.ShapeDtypeStruct((B,), jnp.float32),
        scratch_types=[
            pltpu.VMEM((b_per_w,), jnp.int32),
            pltpu.VMEM((b_per_w,), jnp.int32),
            pltpu.VMEM((b_per_w,), jnp.int32),
            pltpu.VMEM((b_per_w,), jnp.int32),
            pltpu.VMEM((2, chunk, _SR), jnp.float32),
            pltpu.VMEM((2, chunk, _SR), jnp.float32),
            pltpu.VMEM((b_per_w,), jnp.float32),
            pltpu.SemaphoreType.DMA,
            pltpu.SemaphoreType.DMA,
        ],
    )
    def k(c1_hbm, c2_hbm, t_hbm, out_hbm,
          idx1_v, idx2_v, sr1_v, sr2_v, r1_v, r2_v, out_v, sem_a, sem_b):
        wid = lax.axis_index("s") * info.num_cores + lax.axis_index("c")
        base = wid * b_per_w
        pltpu.sync_copy(c1_hbm.at[pl.ds(base, b_per_w)], idx1_v)
        pltpu.sync_copy(c2_hbm.at[pl.ds(base, b_per_w)], idx2_v)

        lane = lax.iota(jnp.int32, _L)
        n_groups = chunk // _L

        def srow(g, carry):
            sl = pl.ds(g * _L, _L)
            sr1_v[sl] = idx1_v[sl] & (_QS - 1)
            sr2_v[sl] = idx2_v[sl] & (_QS - 1)
            return carry

        lax.fori_loop(0, b_per_w // _L, srow, 0)

        sems = (sem_a, sem_b)

        def fire(h):
            hb = pl.ds(h * chunk, chunk)
            sem = sems[h % 2]
            return (
                pltpu.async_copy(t_hbm.at[sr1_v.at[hb]], r1_v.at[h % 2], sem),
                pltpu.async_copy(t_hbm.at[sr2_v.at[hb]], r2_v.at[h % 2], sem),
            )

        def compute(h):
            def group(g, carry):
                src = pl.ds(h * chunk + g * _L, _L)
                rows = lane + g * _L
                o1 = jax.lax.shift_left(
                    jax.lax.shift_right_logical(idx1_v[src], 18), 5)
                o2 = jax.lax.shift_left(
                    jax.lax.shift_right_logical(idx2_v[src], 18), 5)
                acc = jnp.zeros((_L,), jnp.float32)
                for d in range(_D):
                    a = plsc.load_gather(r1_v.at[h % 2], [rows, o1 + d])
                    b = plsc.load_gather(r2_v.at[h % 2], [rows, o2 + d])
                    acc = acc + a * b
                out_v[src] = acc
                return carry

            lax.fori_loop(0, n_groups, group, 0)

        n_chunks = b_per_w // chunk
        cps = fire(0)
        for h in range(n_chunks):
            for cp in cps:
                cp.wait()
            if h + 1 < n_chunks:
                cps = fire(h + 1)
            compute(h)

        pltpu.sync_copy(out_v, out_hbm.at[pl.ds(base, b_per_w)])

    return k(champ1.astype(jnp.int32), champ2.astype(jnp.int32), t128)


def kernel(champ1, champ2, table):
    return _run(champ1, champ2, table).reshape(-1, 1, 1)
